# Initial kernel scaffold; baseline (speedup 1.0000x reference)
import jax
import jax.numpy as jnp
from jax.experimental import pallas as pl


def kernel(x, edge_index, edge_attr, batch, W0, b0, We1, be1, We2, be2, b_conv, W_ih, W_hh, b_ih, b_hh, Wl_ih, Wl_hh, bl_ih, bl_hh, Wm1, bm1, Wm2, bm2):
    raise NotImplementedError("write your pallas kernel here")


# speedup vs baseline: 1.0000x; 1.0000x over previous
import jax
import jax.numpy as jnp
from jax.experimental import pallas as pl

N = 10000
E = 160000
DIM = 32
B = 128


def kernel(x, edge_index, edge_attr, batch, W0, b0, We1, be1, We2, be2, b_conv,
           W_ih, W_hh, b_ih, b_hh, Wl_ih, Wl_hh, bl_ih, bl_hh, Wm1, bm1, Wm2, bm2):
    src = edge_index[0]
    dst = edge_index[1]
    deg = jnp.clip(jax.ops.segment_sum(jnp.ones((E,), jnp.float32), dst, num_segments=N), 1.0, None)
    out = jax.nn.relu(x @ W0 + b0)
    h = out
    for _ in range(3):
        We = jax.nn.relu(edge_attr @ We1 + be1) @ We2 + be2
        We = We.reshape(E, DIM, DIM)
        msg = jnp.einsum('ei,eio->eo', out[src], We)
        agg = jax.ops.segment_sum(msg, dst, num_segments=N) / deg[:, None]
        m = jax.nn.relu(agg + b_conv)
        gi = m @ W_ih.T + b_ih
        gh = h @ W_hh.T + b_hh
        ir, iz, inn = jnp.split(gi, 3, axis=1)
        hr, hz, hn = jnp.split(gh, 3, axis=1)
        r = jax.nn.sigmoid(ir + hr)
        z = jax.nn.sigmoid(iz + hz)
        n = jnp.tanh(inn + r * hn)
        h = (1.0 - z) * n + z * h
        out = h
    q_star = jnp.zeros((B, 2 * DIM), jnp.float32)
    hl = jnp.zeros((B, DIM), jnp.float32)
    cl = jnp.zeros((B, DIM), jnp.float32)
    for _ in range(3):
        gates = q_star @ Wl_ih.T + bl_ih + hl @ Wl_hh.T + bl_hh
        ig, fg, gg, og = jnp.split(gates, 4, axis=1)
        ig = jax.nn.sigmoid(ig)
        fg = jax.nn.sigmoid(fg)
        gg = jnp.tanh(gg)
        og = jax.nn.sigmoid(og)
        cl = fg * cl + ig * gg
        hl = og * jnp.tanh(cl)
        q = hl
        e = jnp.sum(out * q[batch], axis=1)
        seg_max = jax.lax.stop_gradient(jax.ops.segment_max(e, batch, num_segments=B))
        a_un = jnp.exp(e - seg_max[batch])
        denom = jax.ops.segment_sum(a_un, batch, num_segments=B)
        a = a_un / denom[batch]
        r2 = jax.ops.segment_sum(a[:, None] * out, batch, num_segments=B)
        q_star = jnp.concatenate([q, r2], axis=1)
    pred = jax.nn.relu(q_star @ Wm1 + bm1) @ Wm2 + bm2
    return pred.reshape(-1)


# speedup vs baseline: 1.7380x; 1.7380x over previous
import jax
import jax.numpy as jnp
from jax import lax
from jax.experimental import pallas as pl
from jax.experimental.pallas import tpu as pltpu
from jax.experimental.pallas import tpu_sc as plsc

N = 10000
E = 160000
F_IN = 128
DIM = 32
B = 128
H = 128
PAD = 128

NC = 2
NS = 16
NW = NC * NS
CH = 128
NCHUNK = E // CH
FULL = NCHUNK // NW
TAIL = NCHUNK - FULL * NW

BE = 1280
NBLK = 2000

_f32 = jnp.float32
_PH = lax.Precision.HIGHEST


def _lin0_body(x_ref, w_ref, b_ref, o_ref):
    o = jax.nn.relu(
        jnp.dot(x_ref[...], w_ref[...], preferred_element_type=_f32, precision=_PH) + b_ref[...]
    )
    o_ref[...] = jnp.concatenate([o, jnp.zeros((N, PAD - DIM), _f32)], axis=1)


def _lin0(x, W0, b0row):
    return pl.pallas_call(
        _lin0_body,
        out_shape=jax.ShapeDtypeStruct((N, PAD), _f32),
    )(x, W0, b0row)


def _msg_body(ea_ref, gs_ref, we1t_ref, be1_ref, wbig_ref, be2_ref, o_ref):
    ea = ea_ref[...]
    h2t = jax.nn.relu(
        jnp.dot(we1t_ref[...], ea, preferred_element_type=_f32, precision=_PH) + be1_ref[...]
    )
    at = jnp.dot(wbig_ref[...], h2t, preferred_element_type=_f32, precision=_PH) + be2_ref[...]
    a3 = at.reshape(DIM, DIM, BE)
    outt = gs_ref[:, 0:DIM].T
    msgt = jnp.sum(a3 * outt[:, None, :], axis=0)
    o_ref[...] = jnp.concatenate(
        [msgt.T, jnp.zeros((BE, PAD - DIM), _f32)], axis=1)


def _msg(eaT, gsrc, We1T, be1col, Wbig, be2col):
    grid = (E // BE,)
    return pl.pallas_call(
        _msg_body,
        grid=grid,
        in_specs=[
            pl.BlockSpec((5, BE), lambda i: (0, i)),
            pl.BlockSpec((BE, PAD), lambda i: (i, 0)),
            pl.BlockSpec((H, 5), lambda i: (0, 0)),
            pl.BlockSpec((H, 1), lambda i: (0, 0)),
            pl.BlockSpec((DIM * DIM, H), lambda i: (0, 0)),
            pl.BlockSpec((DIM * DIM, 1), lambda i: (0, 0)),
        ],
        out_specs=pl.BlockSpec((BE, PAD), lambda i: (i, 0)),
        out_shape=jax.ShapeDtypeStruct((E, PAD), _f32),
    )(eaT, gsrc, We1T, be1col, Wbig, be2col)


def _gru_body(p_ref, d_ref, h_ref, wir_ref, wiz_ref, win_ref,
              whr_ref, whz_ref, whn_ref, bi_ref, bh_ref, bc_ref, o_ref):
    agg = p_ref[0][:, 0:DIM] + p_ref[1][:, 0:DIM]
    deg = jnp.maximum(d_ref[0] + d_ref[1], 1.0)
    m = jax.nn.relu(agg / deg + bc_ref[...])
    h = h_ref[:, 0:DIM]
    ir = jnp.dot(m, wir_ref[...], preferred_element_type=_f32, precision=_PH) + bi_ref[0:1, :]
    iz = jnp.dot(m, wiz_ref[...], preferred_element_type=_f32, precision=_PH) + bi_ref[1:2, :]
    inn = jnp.dot(m, win_ref[...], preferred_element_type=_f32, precision=_PH) + bi_ref[2:3, :]
    hr = jnp.dot(h, whr_ref[...], preferred_element_type=_f32, precision=_PH) + bh_ref[0:1, :]
    hz = jnp.dot(h, whz_ref[...], preferred_element_type=_f32, precision=_PH) + bh_ref[1:2, :]
    hn = jnp.dot(h, whn_ref[...], preferred_element_type=_f32, precision=_PH) + bh_ref[2:3, :]
    r = jax.nn.sigmoid(ir + hr)
    z = jax.nn.sigmoid(iz + hz)
    n = jnp.tanh(inn + r * hn)
    hnew = (1.0 - z) * n + z * h
    o_ref[...] = jnp.concatenate(
        [hnew, jnp.zeros((NBLK, PAD - DIM), _f32)], axis=1)


def _gru(parts, deg1, h, Wi3, Wh3, bi3, bh3, bcrow):
    grid = (N // NBLK,)
    wspec = pl.BlockSpec((DIM, DIM), lambda i: (0, 0))
    return pl.pallas_call(
        _gru_body,
        grid=grid,
        in_specs=[
            pl.BlockSpec((2, NBLK, PAD), lambda i: (0, i, 0)),
            pl.BlockSpec((2, NBLK, 1), lambda i: (0, i, 0)),
            pl.BlockSpec((NBLK, PAD), lambda i: (i, 0)),
            wspec, wspec, wspec, wspec, wspec, wspec,
            pl.BlockSpec((3, DIM), lambda i: (0, 0)),
            pl.BlockSpec((3, DIM), lambda i: (0, 0)),
            pl.BlockSpec((1, DIM), lambda i: (0, 0)),
        ],
        out_specs=pl.BlockSpec((NBLK, PAD), lambda i: (i, 0)),
        out_shape=jax.ShapeDtypeStruct((N, PAD), _f32),
    )(parts, deg1, h, Wi3[0], Wi3[1], Wi3[2], Wh3[0], Wh3[1], Wh3[2],
      bi3, bh3, bcrow)


_S2S_CNK = 2000


def _s2s_chunk_mask(b_ref, c):
    ids = b_ref[pl.ds(c * _S2S_CNK, _S2S_CNK), :]
    cols = lax.broadcasted_iota(jnp.int32, (_S2S_CNK, B), 1)
    return (ids == cols).astype(_f32)


def _s2s_body(out_ref, b_ref, wli_ref, wlh_ref, bl_ref,
              wm1_ref, bm1_ref, wm2_ref, bm2_ref, o_ref):
    nch = N // _S2S_CNK

    q_star = jnp.zeros((B, 2 * DIM), _f32)
    hl = jnp.zeros((B, DIM), _f32)
    cl = jnp.zeros((B, DIM), _f32)
    for _ in range(3):
        gates = (
            jnp.dot(q_star, wli_ref[...], preferred_element_type=_f32, precision=_PH)
            + jnp.dot(hl, wlh_ref[...], preferred_element_type=_f32, precision=_PH)
            + bl_ref[...]
        )
        ig = jax.nn.sigmoid(gates[:, 0 * DIM:1 * DIM])
        fg = jax.nn.sigmoid(gates[:, 1 * DIM:2 * DIM])
        gg = jnp.tanh(gates[:, 2 * DIM:3 * DIM])
        og = jax.nn.sigmoid(gates[:, 3 * DIM:4 * DIM])
        cl = fg * cl + ig * gg
        hl = og * jnp.tanh(cl)
        q = hl

        def _e_chunk(c, mask):
            outc = out_ref[pl.ds(c * _S2S_CNK, _S2S_CNK), 0:DIM]
            qrow = jnp.dot(mask, q, preferred_element_type=_f32, precision=_PH)
            e = jnp.sum(outc * qrow, axis=1, keepdims=True)
            return outc, e

        def _p1(c, smax):
            mask = _s2s_chunk_mask(b_ref, c)
            _, e = _e_chunk(c, mask)
            e_b = jnp.where(mask > 0.0, e, -1e30)
            return jnp.maximum(smax, jnp.max(e_b, axis=0, keepdims=True))

        segmax = lax.fori_loop(0, nch, _p1, jnp.full((1, B), -1e30, _f32))

        def _p2(c, acc):
            mask = _s2s_chunk_mask(b_ref, c)
            outc, e = _e_chunk(c, mask)
            mx = jnp.sum(mask * segmax, axis=1, keepdims=True)
            a_un = jnp.exp(e - mx)
            rhs = jnp.concatenate([a_un * outc, a_un], axis=1)
            return acc + lax.dot_general(
                mask, rhs, (((0,), (0,)), ((), ())),
                preferred_element_type=_f32, precision=_PH)

        acc = lax.fori_loop(0, nch, _p2, jnp.zeros((B, DIM + 1), _f32))
        r2 = acc[:, 0:DIM] / acc[:, DIM:DIM + 1]
        q_star = jnp.concatenate([q, r2], axis=1)

    t = jax.nn.relu(
        jnp.dot(q_star, wm1_ref[...], preferred_element_type=_f32, precision=_PH) + bm1_ref[...]
    )
    o_ref[...] = jnp.dot(t, wm2_ref[...], preferred_element_type=_f32, precision=_PH) + bm2_ref[...]


def _s2s(out, batch2d, Wl_ihT, Wl_hhT, blrow, Wm1, bm1row, Wm2, bm2row):
    return pl.pallas_call(
        _s2s_body,
        out_shape=jax.ShapeDtypeStruct((B, 1), _f32),
    )(out, batch2d, Wl_ihT, Wl_hhT, blrow, Wm1, bm1row, Wm2, bm2row)


def _sc_mesh():
    return plsc.VectorSubcoreMesh(core_axis_name="c", subcore_axis_name="s")


def _gather_body(nodes_hbm, idx_hbm, dep_hbm, o_hbm, idx_v, rows_v, sem):
    c = lax.axis_index("c")
    s = lax.axis_index("s")
    wid = s * NC + c

    def chunk(ci):
        pltpu.sync_copy(idx_hbm.at[ci], idx_v)
        pltpu.async_copy(nodes_hbm.at[idx_v], rows_v, sem).wait()
        pltpu.sync_copy(rows_v, o_hbm.at[pl.ds(ci * CH, CH)])

    def body(j, carry):
        chunk(wid + j * NW)
        return carry

    lax.fori_loop(0, FULL, body, 0)

    @pl.when(wid < TAIL)
    def _():
        chunk(FULL * NW + wid)


def _gather(nodes, idx2, dep):
    k = pl.kernel(
        _gather_body,
        out_type=jax.ShapeDtypeStruct((E, PAD), _f32),
        mesh=_sc_mesh(),
        scratch_types=[
            pltpu.VMEM((CH,), jnp.int32),
            pltpu.VMEM((CH, PAD), _f32),
            pltpu.SemaphoreType.DMA,
        ],
    )
    return k(nodes, idx2, dep)


def _scatter_body(vals_hbm, idx_hbm, zero_hbm, o_hbm, idx_v, upd_v, accsh, sem):
    c = lax.axis_index("c")
    s = lax.axis_index("s")
    wid = s * NC + c

    @pl.when(s == 0)
    def _():
        pltpu.sync_copy(zero_hbm, accsh)

    plsc.subcore_barrier()

    def chunk(ci):
        pltpu.sync_copy(idx_hbm.at[ci], idx_v)
        pltpu.sync_copy(vals_hbm.at[pl.ds(ci * CH, CH)], upd_v)
        pltpu.sync_copy(upd_v, accsh.at[idx_v], add=True)

    def body(j, carry):
        chunk(wid + j * NW)
        return carry

    lax.fori_loop(0, FULL, body, 0)

    @pl.when(wid < TAIL)
    def _():
        chunk(FULL * NW + wid)

    plsc.subcore_barrier()

    @pl.when(s == 0)
    def _():
        pltpu.sync_copy(accsh, o_hbm.at[c])


def _scatter(vals, idx2, zeroN):
    k = pl.kernel(
        _scatter_body,
        out_type=jax.ShapeDtypeStruct((NC, N, PAD), _f32),
        mesh=_sc_mesh(),
        scratch_types=[
            pltpu.VMEM((CH,), jnp.int32),
            pltpu.VMEM((CH, PAD), _f32),
            pltpu.VMEM_SHARED((N, PAD), _f32),
            pltpu.SemaphoreType.DMA,
        ],
    )
    return k(vals, idx2, zeroN)


def _deg_body(e0_hbm, idx_hbm, zero_hbm, o_hbm, idx_v, upd_v, accsh, sem):
    c = lax.axis_index("c")
    s = lax.axis_index("s")
    wid = s * NC + c

    @pl.when(s == 0)
    def _():
        pltpu.sync_copy(zero_hbm, accsh)

    pltpu.sync_copy(e0_hbm, upd_v)
    plsc.subcore_barrier()

    def chunk(ci):
        pltpu.sync_copy(idx_hbm.at[ci], idx_v)
        pltpu.sync_copy(upd_v, accsh.at[idx_v], add=True)

    def body(j, carry):
        chunk(wid + j * NW)
        return carry

    lax.fori_loop(0, FULL, body, 0)

    @pl.when(wid < TAIL)
    def _():
        chunk(FULL * NW + wid)

    plsc.subcore_barrier()

    @pl.when(s == 0)
    def _():
        pltpu.sync_copy(accsh, o_hbm.at[c])


def _deg(e0, idx2, zeroN):
    k = pl.kernel(
        _deg_body,
        out_type=jax.ShapeDtypeStruct((NC, N, PAD), _f32),
        mesh=_sc_mesh(),
        scratch_types=[
            pltpu.VMEM((CH,), jnp.int32),
            pltpu.VMEM((CH, PAD), _f32),
            pltpu.VMEM_SHARED((N, PAD), _f32),
            pltpu.SemaphoreType.DMA,
        ],
    )
    return k(e0, idx2, zeroN)


def kernel(x, edge_index, edge_attr, batch, W0, b0, We1, be1, We2, be2, b_conv,
           W_ih, W_hh, b_ih, b_hh, Wl_ih, Wl_hh, bl_ih, bl_hh, Wm1, bm1, Wm2, bm2):
    src2 = edge_index[0].reshape(NCHUNK, CH)
    dst2 = edge_index[1].reshape(NCHUNK, CH)
    eaT = edge_attr.T
    zeroN = jnp.zeros((N, PAD), _f32)
    e0 = jnp.zeros((CH, PAD), _f32).at[:, 0].set(1.0)

    We1T = We1.T
    be1col = be1.reshape(H, 1)
    Wbig = We2.T
    be2col = be2.reshape(DIM * DIM, 1)
    Wi3 = W_ih.reshape(3, DIM, DIM).transpose(0, 2, 1)
    Wh3 = W_hh.reshape(3, DIM, DIM).transpose(0, 2, 1)
    bi3 = b_ih.reshape(3, DIM)
    bh3 = b_hh.reshape(3, DIM)
    bcrow = b_conv.reshape(1, DIM)
    b0row = b0.reshape(1, DIM)
    batch2d = batch.reshape(N, 1).astype(jnp.int32)
    Wl_ihT = Wl_ih.T
    Wl_hhT = Wl_hh.T
    blrow = (bl_ih + bl_hh).reshape(1, 4 * DIM)
    bm1row = bm1.reshape(1, DIM)
    bm2row = bm2.reshape(1, 1)

    out = _lin0(x, W0, b0row)
    degp = _deg(e0, dst2, zeroN)
    deg1 = degp[:, :, 0:1]

    h = out
    for _ in range(3):
        gsrc = _gather(out, src2, degp)
        msg = _msg(eaT, gsrc, We1T, be1col, Wbig, be2col)
        parts = _scatter(msg, dst2, zeroN)
        h = _gru(parts, deg1, h, Wi3, Wh3, bi3, bh3, bcrow)
        out = h

    pred = _s2s(out, batch2d, Wl_ihT, Wl_hhT, blrow, Wm1, bm1row, Wm2, bm2row)
    return pred.reshape(-1)


# speedup vs baseline: 3.3963x; 1.9541x over previous
import jax
import jax.numpy as jnp
from jax import lax
from jax.experimental import pallas as pl
from jax.experimental.pallas import tpu as pltpu
from jax.experimental.pallas import tpu_sc as plsc

N = 10000
E = 160000
F_IN = 128
DIM = 32
B = 128
H = 128
PAD = 128

NC = 2
NS = 16
NW = NC * NS
CH = 128
NCHUNK = E // CH
FULL = NCHUNK // NW
TAIL = NCHUNK - FULL * NW

BE = 1280
NBLK = 2000

_f32 = jnp.float32
_PH = lax.Precision.HIGHEST
_bf16 = jnp.bfloat16


def _dotbf(a, b):
    return jnp.dot(a.astype(_bf16), b.astype(_bf16), preferred_element_type=_f32)


def _lin0_body(x_ref, w_ref, b_ref, o_ref):
    o = jax.nn.relu(_dotbf(x_ref[...], w_ref[...]) + b_ref[...])
    o_ref[...] = jnp.concatenate([o, jnp.zeros((N, PAD - DIM), _f32)], axis=1)


def _lin0(x, W0, b0row):
    return pl.pallas_call(
        _lin0_body,
        out_shape=jax.ShapeDtypeStruct((N, PAD), _f32),
    )(x, W0, b0row)


def _msg_body(ea_ref, gs_ref, we1t_ref, be1_ref, wbig_ref, be2_ref, o_ref):
    ea = ea_ref[...]
    h2t = jax.nn.relu(_dotbf(we1t_ref[...], ea) + be1_ref[...])
    at = _dotbf(wbig_ref[...], h2t) + be2_ref[...]
    a3 = at.astype(_bf16).astype(_f32).reshape(DIM, DIM, BE)
    outt = gs_ref[:, 0:DIM].astype(_bf16).astype(_f32).T
    msgt = jnp.sum(a3 * outt[:, None, :], axis=0)
    o_ref[...] = jnp.concatenate(
        [msgt.T, jnp.zeros((BE, PAD - DIM), _f32)], axis=1)


def _msg(eaT, gsrc, We1T, be1col, Wbig, be2col):
    grid = (E // BE,)
    return pl.pallas_call(
        _msg_body,
        grid=grid,
        in_specs=[
            pl.BlockSpec((5, BE), lambda i: (0, i)),
            pl.BlockSpec((BE, PAD), lambda i: (i, 0)),
            pl.BlockSpec((H, 5), lambda i: (0, 0)),
            pl.BlockSpec((H, 1), lambda i: (0, 0)),
            pl.BlockSpec((DIM * DIM, H), lambda i: (0, 0)),
            pl.BlockSpec((DIM * DIM, 1), lambda i: (0, 0)),
        ],
        out_specs=pl.BlockSpec((BE, PAD), lambda i: (i, 0)),
        out_shape=jax.ShapeDtypeStruct((E, PAD), _f32),
    )(eaT, gsrc, We1T, be1col, Wbig, be2col)


def _gru_body(p_ref, d_ref, h_ref, wir_ref, wiz_ref, win_ref,
              whr_ref, whz_ref, whn_ref, bi_ref, bh_ref, bc_ref, o_ref):
    agg = p_ref[0][:, 0:DIM] + p_ref[1][:, 0:DIM]
    deg = jnp.maximum(d_ref[0] + d_ref[1], 1.0)
    m = jax.nn.relu(agg / deg + bc_ref[...])
    h = h_ref[:, 0:DIM]
    ir = _dotbf(m, wir_ref[...]) + bi_ref[0:1, :]
    iz = _dotbf(m, wiz_ref[...]) + bi_ref[1:2, :]
    inn = _dotbf(m, win_ref[...]) + bi_ref[2:3, :]
    hr = _dotbf(h, whr_ref[...]) + bh_ref[0:1, :]
    hz = _dotbf(h, whz_ref[...]) + bh_ref[1:2, :]
    hn = _dotbf(h, whn_ref[...]) + bh_ref[2:3, :]
    r = jax.nn.sigmoid(ir + hr)
    z = jax.nn.sigmoid(iz + hz)
    n = jnp.tanh(inn + r * hn)
    hnew = (1.0 - z) * n + z * h
    o_ref[...] = jnp.concatenate(
        [hnew, jnp.zeros((NBLK, PAD - DIM), _f32)], axis=1)


def _gru(parts, deg1, h, Wi3, Wh3, bi3, bh3, bcrow):
    grid = (N // NBLK,)
    wspec = pl.BlockSpec((DIM, DIM), lambda i: (0, 0))
    return pl.pallas_call(
        _gru_body,
        grid=grid,
        in_specs=[
            pl.BlockSpec((2, NBLK, PAD), lambda i: (0, i, 0)),
            pl.BlockSpec((2, NBLK, 1), lambda i: (0, i, 0)),
            pl.BlockSpec((NBLK, PAD), lambda i: (i, 0)),
            wspec, wspec, wspec, wspec, wspec, wspec,
            pl.BlockSpec((3, DIM), lambda i: (0, 0)),
            pl.BlockSpec((3, DIM), lambda i: (0, 0)),
            pl.BlockSpec((1, DIM), lambda i: (0, 0)),
        ],
        out_specs=pl.BlockSpec((NBLK, PAD), lambda i: (i, 0)),
        out_shape=jax.ShapeDtypeStruct((N, PAD), _f32),
    )(parts, deg1, h, Wi3[0], Wi3[1], Wi3[2], Wh3[0], Wh3[1], Wh3[2],
      bi3, bh3, bcrow)


_S2S_CNK = 2000


def _s2s_chunk_mask(b_ref, c):
    ids = b_ref[pl.ds(c * _S2S_CNK, _S2S_CNK), :]
    cols = lax.broadcasted_iota(jnp.int32, (_S2S_CNK, B), 1)
    return (ids == cols).astype(_f32)


def _s2s_body(out_ref, b_ref, wli_ref, wlh_ref, bl_ref,
              wm1_ref, bm1_ref, wm2_ref, bm2_ref, o_ref):
    nch = N // _S2S_CNK

    q_star = jnp.zeros((B, 2 * DIM), _f32)
    hl = jnp.zeros((B, DIM), _f32)
    cl = jnp.zeros((B, DIM), _f32)
    for _ in range(3):
        gates = (
            _dotbf(q_star, wli_ref[...])
            + _dotbf(hl, wlh_ref[...])
            + bl_ref[...]
        )
        ig = jax.nn.sigmoid(gates[:, 0 * DIM:1 * DIM])
        fg = jax.nn.sigmoid(gates[:, 1 * DIM:2 * DIM])
        gg = jnp.tanh(gates[:, 2 * DIM:3 * DIM])
        og = jax.nn.sigmoid(gates[:, 3 * DIM:4 * DIM])
        cl = fg * cl + ig * gg
        hl = og * jnp.tanh(cl)
        q = hl

        def _e_chunk(c, mask):
            outc = out_ref[pl.ds(c * _S2S_CNK, _S2S_CNK), 0:DIM]
            qrow = jnp.dot(mask, q, preferred_element_type=_f32, precision=_PH)
            e = jnp.sum(outc * qrow, axis=1, keepdims=True)
            return outc, e

        def _p1(c, smax):
            mask = _s2s_chunk_mask(b_ref, c)
            _, e = _e_chunk(c, mask)
            e_b = jnp.where(mask > 0.0, e, -1e30)
            return jnp.maximum(smax, jnp.max(e_b, axis=0, keepdims=True))

        segmax = lax.fori_loop(0, nch, _p1, jnp.full((1, B), -1e30, _f32))

        def _p2(c, acc):
            mask = _s2s_chunk_mask(b_ref, c)
            outc, e = _e_chunk(c, mask)
            mx = jnp.sum(mask * segmax, axis=1, keepdims=True)
            a_un = jnp.exp(e - mx)
            rhs = jnp.concatenate([a_un * outc, a_un], axis=1)
            return acc + lax.dot_general(
                mask, rhs, (((0,), (0,)), ((), ())),
                preferred_element_type=_f32, precision=_PH)

        acc = lax.fori_loop(0, nch, _p2, jnp.zeros((B, DIM + 1), _f32))
        r2 = acc[:, 0:DIM] / acc[:, DIM:DIM + 1]
        q_star = jnp.concatenate([q, r2], axis=1)

    t = jax.nn.relu(
        _dotbf(q_star, wm1_ref[...]) + bm1_ref[...]
    )
    o_ref[...] = _dotbf(t, wm2_ref[...]) + bm2_ref[...]


def _s2s(out, batch2d, Wl_ihT, Wl_hhT, blrow, Wm1, bm1row, Wm2, bm2row):
    return pl.pallas_call(
        _s2s_body,
        out_shape=jax.ShapeDtypeStruct((B, 1), _f32),
    )(out, batch2d, Wl_ihT, Wl_hhT, blrow, Wm1, bm1row, Wm2, bm2row)


def _sc_mesh():
    return plsc.VectorSubcoreMesh(core_axis_name="c", subcore_axis_name="s")


def _gather_body(nodes_hbm, idx_hbm, dep_hbm, o_hbm, idx_v, rows_v, sem):
    c = lax.axis_index("c")
    s = lax.axis_index("s")
    wid = s * NC + c

    def chunk(ci):
        pltpu.sync_copy(idx_hbm.at[ci], idx_v)
        pltpu.async_copy(nodes_hbm.at[idx_v], rows_v, sem).wait()
        pltpu.sync_copy(rows_v, o_hbm.at[pl.ds(ci * CH, CH)])

    def body(j, carry):
        chunk(wid + j * NW)
        return carry

    lax.fori_loop(0, FULL, body, 0)

    @pl.when(wid < TAIL)
    def _():
        chunk(FULL * NW + wid)


def _gather(nodes, idx2, dep):
    k = pl.kernel(
        _gather_body,
        out_type=jax.ShapeDtypeStruct((E, PAD), _f32),
        mesh=_sc_mesh(),
        scratch_types=[
            pltpu.VMEM((CH,), jnp.int32),
            pltpu.VMEM((CH, PAD), _f32),
            pltpu.SemaphoreType.DMA,
        ],
    )
    return k(nodes, idx2, dep)


def _scatter_body(vals_hbm, idx_hbm, zero_hbm, o_hbm, idx_v, upd_v, accsh, sem):
    c = lax.axis_index("c")
    s = lax.axis_index("s")
    wid = s * NC + c

    @pl.when(s == 0)
    def _():
        pltpu.sync_copy(zero_hbm, accsh)

    plsc.subcore_barrier()

    def chunk(ci):
        pltpu.sync_copy(idx_hbm.at[ci], idx_v)
        pltpu.sync_copy(vals_hbm.at[pl.ds(ci * CH, CH)], upd_v)
        pltpu.sync_copy(upd_v, accsh.at[idx_v], add=True)

    def body(j, carry):
        chunk(wid + j * NW)
        return carry

    lax.fori_loop(0, FULL, body, 0)

    @pl.when(wid < TAIL)
    def _():
        chunk(FULL * NW + wid)

    plsc.subcore_barrier()

    @pl.when(s == 0)
    def _():
        pltpu.sync_copy(accsh, o_hbm.at[c])


def _scatter(vals, idx2, zeroN):
    k = pl.kernel(
        _scatter_body,
        out_type=jax.ShapeDtypeStruct((NC, N, PAD), _f32),
        mesh=_sc_mesh(),
        scratch_types=[
            pltpu.VMEM((CH,), jnp.int32),
            pltpu.VMEM((CH, PAD), _f32),
            pltpu.VMEM_SHARED((N, PAD), _f32),
            pltpu.SemaphoreType.DMA,
        ],
    )
    return k(vals, idx2, zeroN)


def _deg_body(e0_hbm, idx_hbm, zero_hbm, o_hbm, idx_v, upd_v, accsh, sem):
    c = lax.axis_index("c")
    s = lax.axis_index("s")
    wid = s * NC + c

    @pl.when(s == 0)
    def _():
        pltpu.sync_copy(zero_hbm, accsh)

    pltpu.sync_copy(e0_hbm, upd_v)
    plsc.subcore_barrier()

    def chunk(ci):
        pltpu.sync_copy(idx_hbm.at[ci], idx_v)
        pltpu.sync_copy(upd_v, accsh.at[idx_v], add=True)

    def body(j, carry):
        chunk(wid + j * NW)
        return carry

    lax.fori_loop(0, FULL, body, 0)

    @pl.when(wid < TAIL)
    def _():
        chunk(FULL * NW + wid)

    plsc.subcore_barrier()

    @pl.when(s == 0)
    def _():
        pltpu.sync_copy(accsh, o_hbm.at[c])


def _deg(e0, idx2, zeroN):
    k = pl.kernel(
        _deg_body,
        out_type=jax.ShapeDtypeStruct((NC, N, PAD), _f32),
        mesh=_sc_mesh(),
        scratch_types=[
            pltpu.VMEM((CH,), jnp.int32),
            pltpu.VMEM((CH, PAD), _f32),
            pltpu.VMEM_SHARED((N, PAD), _f32),
            pltpu.SemaphoreType.DMA,
        ],
    )
    return k(e0, idx2, zeroN)


def kernel(x, edge_index, edge_attr, batch, W0, b0, We1, be1, We2, be2, b_conv,
           W_ih, W_hh, b_ih, b_hh, Wl_ih, Wl_hh, bl_ih, bl_hh, Wm1, bm1, Wm2, bm2):
    src2 = edge_index[0].reshape(NCHUNK, CH)
    dst2 = edge_index[1].reshape(NCHUNK, CH)
    eaT = edge_attr.T
    zeroN = jnp.zeros((N, PAD), _f32)
    e0 = jnp.zeros((CH, PAD), _f32).at[:, 0].set(1.0)

    We1T = We1.T
    be1col = be1.reshape(H, 1)
    Wbig = We2.T
    be2col = be2.reshape(DIM * DIM, 1)
    Wi3 = W_ih.reshape(3, DIM, DIM).transpose(0, 2, 1)
    Wh3 = W_hh.reshape(3, DIM, DIM).transpose(0, 2, 1)
    bi3 = b_ih.reshape(3, DIM)
    bh3 = b_hh.reshape(3, DIM)
    bcrow = b_conv.reshape(1, DIM)
    b0row = b0.reshape(1, DIM)
    batch2d = batch.reshape(N, 1).astype(jnp.int32)
    Wl_ihT = Wl_ih.T
    Wl_hhT = Wl_hh.T
    blrow = (bl_ih + bl_hh).reshape(1, 4 * DIM)
    bm1row = bm1.reshape(1, DIM)
    bm2row = bm2.reshape(1, 1)

    out = _lin0(x, W0, b0row)
    degp = _deg(e0, dst2, zeroN)
    deg1 = degp[:, :, 0:1]

    h = out
    for _ in range(3):
        gsrc = _gather(out, src2, degp)
        msg = _msg(eaT, gsrc, We1T, be1col, Wbig, be2col)
        parts = _scatter(msg, dst2, zeroN)
        h = _gru(parts, deg1, h, Wi3, Wh3, bi3, bh3, bcrow)
        out = h

    pred = _s2s(out, batch2d, Wl_ihT, Wl_hhT, blrow, Wm1, bm1row, Wm2, bm2row)
    return pred.reshape(-1)


# speedup vs baseline: 3.9504x; 1.1632x over previous
import jax
import jax.numpy as jnp
from jax import lax
from jax.experimental import pallas as pl
from jax.experimental.pallas import tpu as pltpu
from jax.experimental.pallas import tpu_sc as plsc

N = 10000
E = 160000
F_IN = 128
DIM = 32
B = 128
H = 128
PAD = 128

NC = 2
NS = 16
NW = NC * NS
CH = 128
NCHUNK = E // CH
FULL = NCHUNK // NW
TAIL = NCHUNK - FULL * NW

BE = 1280
NBLK = 2000

_f32 = jnp.float32
_PH = lax.Precision.HIGHEST
_bf16 = jnp.bfloat16


def _dotbf(a, b):
    return jnp.dot(a.astype(_bf16), b.astype(_bf16), preferred_element_type=_f32)


def _lin0_body(x_ref, w_ref, b_ref, o_ref):
    o = jax.nn.relu(_dotbf(x_ref[...], w_ref[...]) + b_ref[...])
    o_ref[...] = jnp.concatenate([o, jnp.zeros((N, PAD - DIM), _f32)], axis=1)


def _lin0(x, W0, b0row):
    return pl.pallas_call(
        _lin0_body,
        out_shape=jax.ShapeDtypeStruct((N, PAD), _f32),
    )(x, W0, b0row)


def _msg_body(ea_ref, gs_ref, we1t_ref, be1_ref, wbig_ref, be2_ref, o_ref):
    ea = ea_ref[...]
    h2t = jax.nn.relu(_dotbf(we1t_ref[...], ea) + be1_ref[...])
    at = _dotbf(wbig_ref[...], h2t) + be2_ref[...]
    a3 = at.astype(_bf16).astype(_f32).reshape(DIM, DIM, BE)
    outt = gs_ref[:, 0:DIM].astype(_bf16).astype(_f32).T
    msgt = jnp.sum(a3 * outt[:, None, :], axis=0)
    o_ref[...] = jnp.concatenate(
        [msgt.T, jnp.zeros((BE, PAD - DIM), _f32)], axis=1)


def _msg(eaT, gsrc, We1T, be1col, Wbig, be2col):
    grid = (E // BE,)
    return pl.pallas_call(
        _msg_body,
        grid=grid,
        in_specs=[
            pl.BlockSpec((5, BE), lambda i: (0, i)),
            pl.BlockSpec((BE, PAD), lambda i: (i, 0)),
            pl.BlockSpec((H, 5), lambda i: (0, 0)),
            pl.BlockSpec((H, 1), lambda i: (0, 0)),
            pl.BlockSpec((DIM * DIM, H), lambda i: (0, 0)),
            pl.BlockSpec((DIM * DIM, 1), lambda i: (0, 0)),
        ],
        out_specs=pl.BlockSpec((BE, PAD), lambda i: (i, 0)),
        out_shape=jax.ShapeDtypeStruct((E, PAD), _f32),
    )(eaT, gsrc, We1T, be1col, Wbig, be2col)


def _gru_body(p_ref, d_ref, h_ref, wir_ref, wiz_ref, win_ref,
              whr_ref, whz_ref, whn_ref, bi_ref, bh_ref, bc_ref, o_ref):
    agg = p_ref[0][:, 0:DIM] + p_ref[1][:, 0:DIM]
    deg = jnp.maximum(d_ref[0] + d_ref[1], 1.0)
    m = jax.nn.relu(agg / deg + bc_ref[...])
    h = h_ref[:, 0:DIM]
    ir = _dotbf(m, wir_ref[...]) + bi_ref[0:1, :]
    iz = _dotbf(m, wiz_ref[...]) + bi_ref[1:2, :]
    inn = _dotbf(m, win_ref[...]) + bi_ref[2:3, :]
    hr = _dotbf(h, whr_ref[...]) + bh_ref[0:1, :]
    hz = _dotbf(h, whz_ref[...]) + bh_ref[1:2, :]
    hn = _dotbf(h, whn_ref[...]) + bh_ref[2:3, :]
    r = jax.nn.sigmoid(ir + hr)
    z = jax.nn.sigmoid(iz + hz)
    n = jnp.tanh(inn + r * hn)
    hnew = (1.0 - z) * n + z * h
    o_ref[...] = jnp.concatenate(
        [hnew, jnp.zeros((NBLK, PAD - DIM), _f32)], axis=1)


def _gru(parts, deg1, h, Wi3, Wh3, bi3, bh3, bcrow):
    grid = (N // NBLK,)
    wspec = pl.BlockSpec((DIM, DIM), lambda i: (0, 0))
    return pl.pallas_call(
        _gru_body,
        grid=grid,
        in_specs=[
            pl.BlockSpec((2, NBLK, PAD), lambda i: (0, i, 0)),
            pl.BlockSpec((2, NBLK, 1), lambda i: (0, i, 0)),
            pl.BlockSpec((NBLK, PAD), lambda i: (i, 0)),
            wspec, wspec, wspec, wspec, wspec, wspec,
            pl.BlockSpec((3, DIM), lambda i: (0, 0)),
            pl.BlockSpec((3, DIM), lambda i: (0, 0)),
            pl.BlockSpec((1, DIM), lambda i: (0, 0)),
        ],
        out_specs=pl.BlockSpec((NBLK, PAD), lambda i: (i, 0)),
        out_shape=jax.ShapeDtypeStruct((N, PAD), _f32),
    )(parts, deg1, h, Wi3[0], Wi3[1], Wi3[2], Wh3[0], Wh3[1], Wh3[2],
      bi3, bh3, bcrow)


_S2S_CNK = 2000


def _s2s_chunk_mask(b_ref, c):
    ids = b_ref[pl.ds(c * _S2S_CNK, _S2S_CNK), :]
    cols = lax.broadcasted_iota(jnp.int32, (_S2S_CNK, B), 1)
    return (ids == cols).astype(_f32)


def _s2s_body(out_ref, b_ref, wli_ref, wlh_ref, bl_ref,
              wm1_ref, bm1_ref, wm2_ref, bm2_ref, o_ref):
    nch = N // _S2S_CNK

    q_star = jnp.zeros((B, 2 * DIM), _f32)
    hl = jnp.zeros((B, DIM), _f32)
    cl = jnp.zeros((B, DIM), _f32)
    for _ in range(3):
        gates = (
            _dotbf(q_star, wli_ref[...])
            + _dotbf(hl, wlh_ref[...])
            + bl_ref[...]
        )
        ig = jax.nn.sigmoid(gates[:, 0 * DIM:1 * DIM])
        fg = jax.nn.sigmoid(gates[:, 1 * DIM:2 * DIM])
        gg = jnp.tanh(gates[:, 2 * DIM:3 * DIM])
        og = jax.nn.sigmoid(gates[:, 3 * DIM:4 * DIM])
        cl = fg * cl + ig * gg
        hl = og * jnp.tanh(cl)
        q = hl

        def _e_chunk(c, mask):
            outc = out_ref[pl.ds(c * _S2S_CNK, _S2S_CNK), 0:DIM]
            qrow = jnp.dot(mask, q, preferred_element_type=_f32, precision=_PH)
            e = jnp.sum(outc * qrow, axis=1, keepdims=True)
            return outc, e

        def _p1(c, smax):
            mask = _s2s_chunk_mask(b_ref, c)
            _, e = _e_chunk(c, mask)
            e_b = jnp.where(mask > 0.0, e, -1e30)
            return jnp.maximum(smax, jnp.max(e_b, axis=0, keepdims=True))

        segmax = lax.fori_loop(0, nch, _p1, jnp.full((1, B), -1e30, _f32))

        def _p2(c, acc):
            mask = _s2s_chunk_mask(b_ref, c)
            outc, e = _e_chunk(c, mask)
            mx = jnp.sum(mask * segmax, axis=1, keepdims=True)
            a_un = jnp.exp(e - mx)
            rhs = jnp.concatenate([a_un * outc, a_un], axis=1)
            return acc + lax.dot_general(
                mask, rhs, (((0,), (0,)), ((), ())),
                preferred_element_type=_f32, precision=_PH)

        acc = lax.fori_loop(0, nch, _p2, jnp.zeros((B, DIM + 1), _f32))
        r2 = acc[:, 0:DIM] / acc[:, DIM:DIM + 1]
        q_star = jnp.concatenate([q, r2], axis=1)

    t = jax.nn.relu(
        _dotbf(q_star, wm1_ref[...]) + bm1_ref[...]
    )
    o_ref[...] = _dotbf(t, wm2_ref[...]) + bm2_ref[...]


def _s2s(out, batch2d, Wl_ihT, Wl_hhT, blrow, Wm1, bm1row, Wm2, bm2row):
    return pl.pallas_call(
        _s2s_body,
        out_shape=jax.ShapeDtypeStruct((B, 1), _f32),
    )(out, batch2d, Wl_ihT, Wl_hhT, blrow, Wm1, bm1row, Wm2, bm2row)


def _sc_mesh():
    return plsc.VectorSubcoreMesh(core_axis_name="c", subcore_axis_name="s")


def _gather_body(nodes_hbm, idx_hbm, dep_hbm, o_hbm, idx_v, rows_v, semi, semg, semw):
    c = lax.axis_index("c")
    s = lax.axis_index("s")
    wid = s * NC + c

    def cid(j):
        return wid + j * NW

    di = [None, None]
    dw = [None, None]
    di[0] = pltpu.async_copy(idx_hbm.at[cid(0)], idx_v.at[0], semi.at[0])
    for j in range(FULL):
        b = j % 2
        di[b].wait()
        if j + 1 < FULL:
            di[1 - b] = pltpu.async_copy(
                idx_hbm.at[cid(j + 1)], idx_v.at[1 - b], semi.at[1 - b])
        if j >= 2:
            dw[b].wait()
        pltpu.async_copy(nodes_hbm.at[idx_v.at[b]], rows_v.at[b], semg).wait()
        dw[b] = pltpu.async_copy(
            rows_v.at[b], o_hbm.at[pl.ds(cid(j) * CH, CH)], semw.at[b])
    dw[(FULL - 1) % 2].wait()
    dw[(FULL - 2) % 2].wait()

    @pl.when(wid < TAIL)
    def _():
        ci = FULL * NW + wid
        pltpu.sync_copy(idx_hbm.at[ci], idx_v.at[0])
        pltpu.async_copy(nodes_hbm.at[idx_v.at[0]], rows_v.at[0], semg).wait()
        pltpu.sync_copy(rows_v.at[0], o_hbm.at[pl.ds(ci * CH, CH)])


def _gather(nodes, idx2, dep):
    k = pl.kernel(
        _gather_body,
        out_type=jax.ShapeDtypeStruct((E, PAD), _f32),
        mesh=_sc_mesh(),
        scratch_types=[
            pltpu.VMEM((2, CH), jnp.int32),
            pltpu.VMEM((2, CH, PAD), _f32),
            pltpu.SemaphoreType.DMA((2,)),
            pltpu.SemaphoreType.DMA,
            pltpu.SemaphoreType.DMA((2,)),
        ],
    )
    return k(nodes, idx2, dep)


def _scatter_body(vals_hbm, idx_hbm, zero_hbm, o_hbm, idx_v, upd_v, accsh, semi, semv):
    c = lax.axis_index("c")
    s = lax.axis_index("s")
    wid = s * NC + c

    @pl.when(s == 0)
    def _():
        pltpu.sync_copy(zero_hbm, accsh)

    plsc.subcore_barrier()

    def cid(j):
        return wid + j * NW

    di = [None, None]
    dv = [None, None]
    di[0] = pltpu.async_copy(idx_hbm.at[cid(0)], idx_v.at[0], semi.at[0])
    dv[0] = pltpu.async_copy(vals_hbm.at[pl.ds(cid(0) * CH, CH)], upd_v.at[0],
                             semv.at[0])
    for j in range(FULL):
        b = j % 2
        di[b].wait()
        dv[b].wait()
        if j + 1 < FULL:
            di[1 - b] = pltpu.async_copy(
                idx_hbm.at[cid(j + 1)], idx_v.at[1 - b], semi.at[1 - b])
            dv[1 - b] = pltpu.async_copy(
                vals_hbm.at[pl.ds(cid(j + 1) * CH, CH)], upd_v.at[1 - b],
                semv.at[1 - b])
        pltpu.sync_copy(upd_v.at[b], accsh.at[idx_v.at[b]], add=True)

    @pl.when(wid < TAIL)
    def _():
        ci = FULL * NW + wid
        pltpu.sync_copy(idx_hbm.at[ci], idx_v.at[0])
        pltpu.sync_copy(vals_hbm.at[pl.ds(ci * CH, CH)], upd_v.at[0])
        pltpu.sync_copy(upd_v.at[0], accsh.at[idx_v.at[0]], add=True)

    plsc.subcore_barrier()

    @pl.when(s == 0)
    def _():
        pltpu.sync_copy(accsh, o_hbm.at[c])


def _scatter(vals, idx2, zeroN):
    k = pl.kernel(
        _scatter_body,
        out_type=jax.ShapeDtypeStruct((NC, N, PAD), _f32),
        mesh=_sc_mesh(),
        scratch_types=[
            pltpu.VMEM((2, CH), jnp.int32),
            pltpu.VMEM((2, CH, PAD), _f32),
            pltpu.VMEM_SHARED((N, PAD), _f32),
            pltpu.SemaphoreType.DMA((2,)),
            pltpu.SemaphoreType.DMA((2,)),
        ],
    )
    return k(vals, idx2, zeroN)


def _deg_body(e0_hbm, idx_hbm, zero_hbm, o_hbm, idx_v, upd_v, accsh, sem):
    c = lax.axis_index("c")
    s = lax.axis_index("s")
    wid = s * NC + c

    @pl.when(s == 0)
    def _():
        pltpu.sync_copy(zero_hbm, accsh)

    pltpu.sync_copy(e0_hbm, upd_v)
    plsc.subcore_barrier()

    def chunk(ci):
        pltpu.sync_copy(idx_hbm.at[ci], idx_v)
        pltpu.sync_copy(upd_v, accsh.at[idx_v], add=True)

    def body(j, carry):
        chunk(wid + j * NW)
        return carry

    lax.fori_loop(0, FULL, body, 0)

    @pl.when(wid < TAIL)
    def _():
        chunk(FULL * NW + wid)

    plsc.subcore_barrier()

    @pl.when(s == 0)
    def _():
        pltpu.sync_copy(accsh, o_hbm.at[c])


def _deg(e0, idx2, zeroN):
    k = pl.kernel(
        _deg_body,
        out_type=jax.ShapeDtypeStruct((NC, N, PAD), _f32),
        mesh=_sc_mesh(),
        scratch_types=[
            pltpu.VMEM((CH,), jnp.int32),
            pltpu.VMEM((CH, PAD), _f32),
            pltpu.VMEM_SHARED((N, PAD), _f32),
            pltpu.SemaphoreType.DMA,
        ],
    )
    return k(e0, idx2, zeroN)


def kernel(x, edge_index, edge_attr, batch, W0, b0, We1, be1, We2, be2, b_conv,
           W_ih, W_hh, b_ih, b_hh, Wl_ih, Wl_hh, bl_ih, bl_hh, Wm1, bm1, Wm2, bm2):
    src2 = edge_index[0].reshape(NCHUNK, CH)
    dst2 = edge_index[1].reshape(NCHUNK, CH)
    eaT = edge_attr.T
    zeroN = jnp.zeros((N, PAD), _f32)
    e0 = jnp.zeros((CH, PAD), _f32).at[:, 0].set(1.0)

    We1T = We1.T
    be1col = be1.reshape(H, 1)
    Wbig = We2.T
    be2col = be2.reshape(DIM * DIM, 1)
    Wi3 = W_ih.reshape(3, DIM, DIM).transpose(0, 2, 1)
    Wh3 = W_hh.reshape(3, DIM, DIM).transpose(0, 2, 1)
    bi3 = b_ih.reshape(3, DIM)
    bh3 = b_hh.reshape(3, DIM)
    bcrow = b_conv.reshape(1, DIM)
    b0row = b0.reshape(1, DIM)
    batch2d = batch.reshape(N, 1).astype(jnp.int32)
    Wl_ihT = Wl_ih.T
    Wl_hhT = Wl_hh.T
    blrow = (bl_ih + bl_hh).reshape(1, 4 * DIM)
    bm1row = bm1.reshape(1, DIM)
    bm2row = bm2.reshape(1, 1)

    out = _lin0(x, W0, b0row)
    degp = _deg(e0, dst2, zeroN)
    deg1 = degp[:, :, 0:1]

    h = out
    for _ in range(3):
        gsrc = _gather(out, src2, degp)
        msg = _msg(eaT, gsrc, We1T, be1col, Wbig, be2col)
        parts = _scatter(msg, dst2, zeroN)
        h = _gru(parts, deg1, h, Wi3, Wh3, bi3, bh3, bcrow)
        out = h

    pred = _s2s(out, batch2d, Wl_ihT, Wl_hhT, blrow, Wm1, bm1row, Wm2, bm2row)
    return pred.reshape(-1)
